# baseline (speedup 1.0000x reference)
import functools

import jax
import jax.numpy as jnp
from jax import lax
from jax.experimental import pallas as pl
from jax.experimental.pallas import tpu as pltpu
from jax.experimental.pallas import tpu_sc as plsc

NC = 2
NS = 16
NW = NC * NS
LANES = 16

BATCH = 4
SEQ_LEN = 4096
D_MODEL = 2048
SC_BATCHES = 1
TC_BATCHES = BATCH - SC_BATCHES
SC_ROWS = SC_BATCHES * SEQ_LEN
ROWS_PER_W = SC_ROWS // NW
TILE_R = 16
N_CHUNKS = ROWS_PER_W // TILE_R


def _sc_body(x_hbm, pos_hbm, out_hbm, pos_v, x_v0, x_v1,
             ld0, ld1, st0, st1):
    wid = lax.axis_index("s") * NC + lax.axis_index("c")
    s0 = wid * ROWS_PER_W

    def seq_row(k):
        return s0 + k * TILE_R

    def x_row(k):
        return TC_BATCHES * SEQ_LEN + seq_row(k)

    def start_load(k, buf, sem):
        pltpu.async_copy(x_hbm.at[pl.ds(x_row(k), TILE_R)], buf, sem)

    def start_store(k, buf, sem):
        pltpu.async_copy(buf, out_hbm.at[pl.ds(seq_row(k), TILE_R)], sem)

    def wait(src, dst, sem):
        pltpu.make_async_copy(src, dst, sem).wait()

    def compute(k, buf):
        pltpu.sync_copy(pos_hbm.at[pl.ds(seq_row(k), TILE_R)], pos_v)

        def row_body(r, _):
            @plsc.parallel_loop(0, D_MODEL, step=LANES, unroll=8)
            def _(j):
                buf[r, pl.ds(j, LANES)] = (
                    buf[r, pl.ds(j, LANES)] + pos_v[r, pl.ds(j, LANES)]
                )

            return 0

        lax.fori_loop(0, TILE_R, row_body, 0)

    start_load(0, x_v0, ld0)
    start_load(1, x_v1, ld1)

    def pair_body(p, _):
        k0 = p * 2
        wait(x_hbm.at[pl.ds(x_row(k0), TILE_R)], x_v0, ld0)
        compute(k0, x_v0)
        start_store(k0, x_v0, st0)

        wait(x_hbm.at[pl.ds(x_row(k0 + 1), TILE_R)], x_v1, ld1)
        compute(k0 + 1, x_v1)
        start_store(k0 + 1, x_v1, st1)

        @pl.when(k0 + 2 < N_CHUNKS)
        def _():
            wait(x_v0, out_hbm.at[pl.ds(seq_row(k0), TILE_R)], st0)
            start_load(k0 + 2, x_v0, ld0)

        @pl.when(k0 + 3 < N_CHUNKS)
        def _():
            wait(x_v1, out_hbm.at[pl.ds(seq_row(k0 + 1), TILE_R)], st1)
            start_load(k0 + 3, x_v1, ld1)

        return 0

    lax.fori_loop(0, N_CHUNKS // 2, pair_body, 0)

    last = N_CHUNKS - 2
    wait(x_v0, out_hbm.at[pl.ds(seq_row(last), TILE_R)], st0)
    wait(x_v1, out_hbm.at[pl.ds(seq_row(last + 1), TILE_R)], st1)


def _sc_add(x2, pos_table):
    k = pl.kernel(
        _sc_body,
        out_type=jax.ShapeDtypeStruct((SC_ROWS, D_MODEL), jnp.float32),
        mesh=plsc.VectorSubcoreMesh(core_axis_name="c", subcore_axis_name="s"),
        scratch_types=[
            pltpu.VMEM((TILE_R, D_MODEL), jnp.float32),
            pltpu.VMEM((TILE_R, D_MODEL), jnp.float32),
            pltpu.VMEM((TILE_R, D_MODEL), jnp.float32),
            pltpu.SemaphoreType.DMA,
            pltpu.SemaphoreType.DMA,
            pltpu.SemaphoreType.DMA,
            pltpu.SemaphoreType.DMA,
        ],
    )
    return k(x2, pos_table)


def _tc_add_kernel(x_ref, pos_ref, o_ref):
    o_ref[...] = x_ref[...] + pos_ref[...][None]


def _tc_add(x, pos_table):
    block_s = 1024
    grid = (SEQ_LEN // block_s, TC_BATCHES)
    return pl.pallas_call(
        _tc_add_kernel,
        grid=grid,
        in_specs=[
            pl.BlockSpec((1, block_s, D_MODEL), lambda j, b: (b, j, 0)),
            pl.BlockSpec((block_s, D_MODEL), lambda j, b: (j, 0)),
        ],
        out_specs=pl.BlockSpec((1, block_s, D_MODEL), lambda j, b: (b, j, 0)),
        out_shape=jax.ShapeDtypeStruct((TC_BATCHES, SEQ_LEN, D_MODEL), x.dtype),
    )(x, pos_table)


def kernel(x, pos_table):
    batch, seq_len, d_model = x.shape
    x2 = x.reshape(batch * seq_len, d_model)
    sc_out = _sc_add(x2, pos_table)
    tc_out = _tc_add(x, pos_table)
    return jnp.concatenate(
        [tc_out, sc_out.reshape(SC_BATCHES, seq_len, d_model)], axis=0
    )


# speedup vs baseline: 2.2065x; 2.2065x over previous
import jax
import jax.numpy as jnp
from jax.experimental import pallas as pl


def _add_pos_kernel(x_ref, pos_ref, o_ref):
    o_ref[...] = x_ref[...] + pos_ref[...][None]


def kernel(x, pos_table):
    batch, seq_len, d_model = x.shape
    block_s = 1024
    while seq_len % block_s:
        block_s //= 2

    grid = (seq_len // block_s, batch)
    return pl.pallas_call(
        _add_pos_kernel,
        grid=grid,
        in_specs=[
            pl.BlockSpec((1, block_s, d_model), lambda j, b: (b, j, 0)),
            pl.BlockSpec((block_s, d_model), lambda j, b: (j, 0)),
        ],
        out_specs=pl.BlockSpec((1, block_s, d_model), lambda j, b: (b, j, 0)),
        out_shape=jax.ShapeDtypeStruct(x.shape, x.dtype),
    )(x, pos_table)
